# baseline (speedup 1.0000x reference)
import functools

import jax
import jax.numpy as jnp
from jax import lax
from jax.experimental import pallas as pl
from jax.experimental.pallas import tpu as pltpu
from jax.experimental.pallas import tpu_sc as plsc

N = 64
S = 64
T = 320
IN_F = 128
HID = 64
MSG = 32
OUT_F = 10
OUT_P = 16
NC = 2
NS = 16
NW = NC * NS
SIMS = S // NW
ROWS = N // NS
L = 16


def _sim_body(x_hbm, nodes_hbm, parents_hbm, first_hbm,
              wenc_hbm, benc_hbm, wns_hbm, bns_hbm, wnm_hbm, bnm_hbm,
              wdec_hbm, bdec_hbm, out_hbm,
              pred0, pred1, msgs0, msgs1, npbuf, nps,
              wns, wnm, bns, bnm, fst,
              xbuf, wenc, benc, wdec, bdec, encstage, fin, enc_sh, dma_sem):
    cid = lax.axis_index("c")
    tid = lax.axis_index("s")
    wid = tid * NC + cid
    s0 = wid * SIMS

    handles = [
        pltpu.async_copy(x_hbm.at[pl.ds(tid * ROWS * IN_F, ROWS * IN_F)],
                         xbuf, dma_sem),
        pltpu.async_copy(wenc_hbm, wenc, dma_sem),
        pltpu.async_copy(benc_hbm, benc, dma_sem),
        pltpu.async_copy(nodes_hbm.at[pl.ds(s0 * T, T)],
                         npbuf.at[pl.ds(0, T)], dma_sem),
        pltpu.async_copy(nodes_hbm.at[pl.ds((s0 + 1) * T, T)],
                         npbuf.at[pl.ds(T, T)], dma_sem),
        pltpu.async_copy(parents_hbm.at[pl.ds(s0 * T, T)],
                         npbuf.at[pl.ds(2 * T, T)], dma_sem),
        pltpu.async_copy(parents_hbm.at[pl.ds((s0 + 1) * T, T)],
                         npbuf.at[pl.ds(3 * T, T)], dma_sem),
        pltpu.async_copy(wns_hbm, wns, dma_sem),
        pltpu.async_copy(wnm_hbm, wnm, dma_sem),
        pltpu.async_copy(bns_hbm, bns, dma_sem),
        pltpu.async_copy(bnm_hbm, bnm, dma_sem),
        pltpu.async_copy(first_hbm, fst, dma_sem),
        pltpu.async_copy(wdec_hbm, wdec, dma_sem),
        pltpu.async_copy(bdec_hbm, bdec, dma_sem),
    ]
    for h in handles:
        h.wait()

    for b in range(4 * T // L):
        v = npbuf[pl.ds(b * L, L)]
        for i in range(L):
            nps[b * L + i] = v[i]

    xv = [[xbuf[pl.ds(r * IN_F + kb * L, L)] for kb in range(IN_F // L)]
          for r in range(ROWS)]
    acc = [[benc[pl.ds(hb * L, L)] for hb in range(4)] for _ in range(ROWS)]
    for k in range(IN_F):
        wcol = [wenc[pl.ds(k * HID + hb * L, L)] for hb in range(4)]
        for r in range(ROWS):
            xk = jnp.broadcast_to(xv[r][k // L][k % L], (L,))
            for hb in range(4):
                acc[r][hb] = acc[r][hb] + wcol[hb] * xk
    for r in range(ROWS):
        for hb in range(4):
            encstage[pl.ds(r * HID + hb * L, L)] = acc[r][hb]

    pltpu.sync_copy(encstage, enc_sh.at[pl.ds(tid * ROWS * HID, ROWS * HID)])
    plsc.subcore_barrier()
    pltpu.sync_copy(enc_sh, pred0)
    pltpu.sync_copy(enc_sh, pred1)

    fstv = [fst[pl.ds(0, L)], fst[pl.ds(L, L)]]

    def step(t, carry):
        nd0 = nps[t]
        nd1 = nps[T + t]
        pa0 = nps[2 * T + t]
        pa1 = nps[3 * T + t]
        uf0 = pa0 < 0
        uf1 = pa1 < 0
        p0 = jnp.maximum(pa0, 0)
        p1 = jnp.maximum(pa1, 0)

        zero = jnp.zeros((L,), jnp.float32)
        a0 = [[bns[pl.ds(hb * L, L)] for hb in range(4)], [zero] * 4]
        a1 = [[bns[pl.ds(hb * L, L)] for hb in range(4)], [zero] * 4]
        m0 = [[bnm[pl.ds(mb * L, L)] for mb in range(2)], [zero] * 2]
        m1 = [[bnm[pl.ds(mb * L, L)] for mb in range(2)], [zero] * 2]

        iv0 = [pred0[pl.ds(nd0 * HID + hb * L, L)] for hb in range(4)]
        iv1 = [pred1[pl.ds(nd1 * HID + hb * L, L)] for hb in range(4)]
        iv0 += [jnp.where(uf0, fstv[mb], msgs0[pl.ds(p0 * MSG + mb * L, L)])
                for mb in range(2)]
        iv1 += [jnp.where(uf1, fstv[mb], msgs1[pl.ds(p1 * MSG + mb * L, L)])
                for mb in range(2)]

        for k in range(HID + MSG):
            v0 = jnp.broadcast_to(iv0[k // L][k % L], (L,))
            v1 = jnp.broadcast_to(iv1[k // L][k % L], (L,))
            par = k % 2
            for hb in range(4):
                w = wns[pl.ds(k * HID + hb * L, L)]
                a0[par][hb] = a0[par][hb] + w * v0
                a1[par][hb] = a1[par][hb] + w * v1
            if k >= HID:
                for mb in range(2):
                    w = wnm[pl.ds(k * MSG + mb * L, L)]
                    m0[par][mb] = m0[par][mb] + w * v0
                    m1[par][mb] = m1[par][mb] + w * v1

        ns0 = [jnp.maximum(a0[0][hb] + a0[1][hb], 0.0) for hb in range(4)]
        ns1 = [jnp.maximum(a1[0][hb] + a1[1][hb], 0.0) for hb in range(4)]
        for hb in range(4):
            pred0[pl.ds(nd0 * HID + hb * L, L)] = ns0[hb]
            pred1[pl.ds(nd1 * HID + hb * L, L)] = ns1[hb]

        for k in range(HID):
            n0 = jnp.broadcast_to(ns0[k // L][k % L], (L,))
            n1 = jnp.broadcast_to(ns1[k // L][k % L], (L,))
            par = k % 2
            for mb in range(2):
                w = wnm[pl.ds(k * MSG + mb * L, L)]
                m0[par][mb] = m0[par][mb] + w * n0
                m1[par][mb] = m1[par][mb] + w * n1
        m0 = [m0[0][mb] + m0[1][mb] for mb in range(2)]
        m1 = [m1[0][mb] + m1[1][mb] for mb in range(2)]

        for mb in range(2):
            msgs0[pl.ds(t * MSG + mb * L, L)] = m0[mb]
            msgs1[pl.ds(t * MSG + mb * L, L)] = m1[mb]
        return carry

    lax.fori_loop(0, T, step, 0, unroll=2)

    for j, (predj, sg) in enumerate(((pred0, s0), (pred1, s0 + 1))):
        fv = [predj[pl.ds(sg * HID + hb * L, L)] for hb in range(4)]
        lg = bdec[pl.ds(0, L)]
        for k in range(HID):
            w = wdec[pl.ds(k * OUT_P, L)]
            lg = lg + w * jnp.broadcast_to(fv[k // L][k % L], (L,))
        ls = [lg[i] for i in range(OUT_F)]
        mx = ls[0]
        for i in range(1, OUT_F):
            mx = jnp.maximum(mx, ls[i])
        sh = lg - jnp.broadcast_to(mx, (L,))
        ex = jnp.exp(sh)
        ssum = ex[0]
        for i in range(1, OUT_F):
            ssum = ssum + ex[i]
        ssv = jnp.broadcast_to(ssum, (L,))
        yv = jnp.full((L,), 1.2, jnp.float32)
        for _ in range(8):
            yv = yv + ssv * jnp.exp(-yv) - 1.0
        fin[pl.ds(j * OUT_P, L)] = sh - yv

    pltpu.sync_copy(fin, out_hbm.at[pl.ds(s0 * OUT_P, SIMS * OUT_P)])


def kernel(x, edge_index, nodes, parents, first_message,
           W_enc, b_enc, W_ns, b_ns, W_nm, b_nm, W_dec, b_dec):
    del edge_index
    f32 = jnp.float32

    sim = pl.kernel(
        _sim_body,
        out_type=jax.ShapeDtypeStruct((S * OUT_P,), f32),
        mesh=plsc.VectorSubcoreMesh(core_axis_name="c", subcore_axis_name="s"),
        scratch_types=[
            pltpu.VMEM((N * HID,), f32),
            pltpu.VMEM((N * HID,), f32),
            pltpu.VMEM((T * MSG,), f32),
            pltpu.VMEM((T * MSG,), f32),
            pltpu.VMEM((4 * T,), jnp.int32),
            pltpu.SMEM((4 * T,), jnp.int32),
            pltpu.VMEM(((HID + MSG) * HID,), f32),
            pltpu.VMEM(((HID + MSG) * MSG,), f32),
            pltpu.VMEM((HID,), f32),
            pltpu.VMEM((MSG,), f32),
            pltpu.VMEM((MSG,), f32),
            pltpu.VMEM((ROWS * IN_F,), f32),
            pltpu.VMEM((IN_F * HID,), f32),
            pltpu.VMEM((HID,), f32),
            pltpu.VMEM((HID * OUT_P,), f32),
            pltpu.VMEM((OUT_P,), f32),
            pltpu.VMEM((ROWS * HID,), f32),
            pltpu.VMEM((SIMS * OUT_P,), f32),
            pltpu.VMEM_SHARED((N * HID,), f32),
            pltpu.SemaphoreType.DMA,
        ],
    )

    wdec_p = jnp.pad(W_dec.T, ((0, 0), (0, OUT_P - OUT_F)))
    bdec_p = jnp.pad(b_dec, (0, OUT_P - OUT_F))
    out = sim(x.reshape(N * IN_F),
              nodes.astype(jnp.int32).reshape(S * T),
              parents.astype(jnp.int32).reshape(S * T),
              first_message.reshape(MSG),
              W_enc.T.reshape(IN_F * HID), b_enc,
              W_ns.T.reshape((HID + MSG) * HID), b_ns,
              W_nm.T.reshape((HID + MSG) * MSG), b_nm,
              wdec_p.reshape(HID * OUT_P), bdec_p)
    return out.reshape(S, OUT_P)[:, :OUT_F]


# speedup vs baseline: 2.0919x; 2.0919x over previous
import functools

import jax
import jax.numpy as jnp
from jax import lax
from jax.experimental import pallas as pl
from jax.experimental.pallas import tpu as pltpu
from jax.experimental.pallas import tpu_sc as plsc

N = 64
S = 64
T = 320
IN_F = 128
HID = 64
MSG = 32
OUT_F = 10
OUT_P = 16
NC = 2
NS = 16
NW = NC * NS
SIMS = S // NW
ROWS = N // NS
L = 16


def _sim_body(x_hbm, nodes_hbm, parents_hbm, first_hbm,
              wenc_hbm, benc_hbm, wns_hbm, bns_hbm, wnm_hbm, bnm_hbm,
              wdec_hbm, bdec_hbm, out_hbm,
              pred0, pred1, msgs0, msgs1, npbuf, nps,
              wns, wnm, bns, bnm, fst,
              xbuf, wenc, benc, wdec, bdec, encstage, fin, enc_sh, dma_sem):
    cid = lax.axis_index("c")
    tid = lax.axis_index("s")
    wid = tid * NC + cid
    s0 = wid * SIMS

    handles = [
        pltpu.async_copy(x_hbm.at[pl.ds(tid * ROWS * IN_F, ROWS * IN_F)],
                         xbuf, dma_sem),
        pltpu.async_copy(wenc_hbm, wenc, dma_sem),
        pltpu.async_copy(benc_hbm, benc, dma_sem),
        pltpu.async_copy(nodes_hbm.at[pl.ds(s0 * T, T)],
                         npbuf.at[pl.ds(0, T)], dma_sem),
        pltpu.async_copy(nodes_hbm.at[pl.ds((s0 + 1) * T, T)],
                         npbuf.at[pl.ds(T, T)], dma_sem),
        pltpu.async_copy(parents_hbm.at[pl.ds(s0 * T, T)],
                         npbuf.at[pl.ds(2 * T, T)], dma_sem),
        pltpu.async_copy(parents_hbm.at[pl.ds((s0 + 1) * T, T)],
                         npbuf.at[pl.ds(3 * T, T)], dma_sem),
        pltpu.async_copy(wns_hbm, wns, dma_sem),
        pltpu.async_copy(wnm_hbm, wnm, dma_sem),
        pltpu.async_copy(bns_hbm, bns, dma_sem),
        pltpu.async_copy(bnm_hbm, bnm, dma_sem),
        pltpu.async_copy(first_hbm, fst, dma_sem),
        pltpu.async_copy(wdec_hbm, wdec, dma_sem),
        pltpu.async_copy(bdec_hbm, bdec, dma_sem),
    ]
    for h in handles:
        h.wait()

    for b in range(4 * T // L):
        v = npbuf[pl.ds(b * L, L)]
        for i in range(L):
            nps[b * L + i] = v[i]

    xv = [[xbuf[pl.ds(r * IN_F + kb * L, L)] for kb in range(IN_F // L)]
          for r in range(ROWS)]
    acc = [[benc[pl.ds(hb * L, L)] for hb in range(4)] for _ in range(ROWS)]
    for k in range(IN_F):
        wcol = [wenc[pl.ds(k * HID + hb * L, L)] for hb in range(4)]
        for r in range(ROWS):
            xk = jnp.broadcast_to(xv[r][k // L][k % L], (L,))
            for hb in range(4):
                acc[r][hb] = acc[r][hb] + wcol[hb] * xk
    for r in range(ROWS):
        for hb in range(4):
            encstage[pl.ds(r * HID + hb * L, L)] = acc[r][hb]

    pltpu.sync_copy(encstage, enc_sh.at[pl.ds(tid * ROWS * HID, ROWS * HID)])
    plsc.subcore_barrier()
    pltpu.sync_copy(enc_sh, pred0)
    pltpu.sync_copy(enc_sh, pred1)

    fstv = [fst[pl.ds(0, L)], fst[pl.ds(L, L)]]

    def step(t, carry):
        nd0 = nps[t]
        nd1 = nps[T + t]
        pa0 = nps[2 * T + t]
        pa1 = nps[3 * T + t]
        uf0 = pa0 < 0
        uf1 = pa1 < 0
        p0 = jnp.maximum(pa0, 0)
        p1 = jnp.maximum(pa1, 0)

        zero = jnp.zeros((L,), jnp.float32)
        a0 = [[bns[pl.ds(hb * L, L)] for hb in range(4)], [zero] * 4]
        a1 = [[bns[pl.ds(hb * L, L)] for hb in range(4)], [zero] * 4]
        m0 = [[bnm[pl.ds(mb * L, L)] for mb in range(2)], [zero] * 2]
        m1 = [[bnm[pl.ds(mb * L, L)] for mb in range(2)], [zero] * 2]

        iv0 = [pred0[pl.ds(nd0 * HID + hb * L, L)] for hb in range(4)]
        iv1 = [pred1[pl.ds(nd1 * HID + hb * L, L)] for hb in range(4)]
        iv0 += [jnp.where(uf0, fstv[mb], msgs0[pl.ds(p0 * MSG + mb * L, L)])
                for mb in range(2)]
        iv1 += [jnp.where(uf1, fstv[mb], msgs1[pl.ds(p1 * MSG + mb * L, L)])
                for mb in range(2)]

        for k in range(HID + MSG):
            v0 = jnp.broadcast_to(iv0[k // L][k % L], (L,))
            v1 = jnp.broadcast_to(iv1[k // L][k % L], (L,))
            par = k % 2
            for hb in range(4):
                w = wns[pl.ds(k * HID + hb * L, L)]
                a0[par][hb] = a0[par][hb] + w * v0
                a1[par][hb] = a1[par][hb] + w * v1
            if k >= HID:
                for mb in range(2):
                    w = wnm[pl.ds(k * MSG + mb * L, L)]
                    m0[par][mb] = m0[par][mb] + w * v0
                    m1[par][mb] = m1[par][mb] + w * v1

        ns0 = [jnp.maximum(a0[0][hb] + a0[1][hb], 0.0) for hb in range(4)]
        ns1 = [jnp.maximum(a1[0][hb] + a1[1][hb], 0.0) for hb in range(4)]
        for hb in range(4):
            pred0[pl.ds(nd0 * HID + hb * L, L)] = ns0[hb]
            pred1[pl.ds(nd1 * HID + hb * L, L)] = ns1[hb]

        for k in range(HID):
            n0 = jnp.broadcast_to(ns0[k // L][k % L], (L,))
            n1 = jnp.broadcast_to(ns1[k // L][k % L], (L,))
            par = k % 2
            for mb in range(2):
                w = wnm[pl.ds(k * MSG + mb * L, L)]
                m0[par][mb] = m0[par][mb] + w * n0
                m1[par][mb] = m1[par][mb] + w * n1
        m0 = [m0[0][mb] + m0[1][mb] for mb in range(2)]
        m1 = [m1[0][mb] + m1[1][mb] for mb in range(2)]

        for mb in range(2):
            msgs0[pl.ds(t * MSG + mb * L, L)] = m0[mb]
            msgs1[pl.ds(t * MSG + mb * L, L)] = m1[mb]
        return carry

    lax.fori_loop(0, T, step, 0)

    for j, (predj, sg) in enumerate(((pred0, s0), (pred1, s0 + 1))):
        fv = [predj[pl.ds(sg * HID + hb * L, L)] for hb in range(4)]
        lg = bdec[pl.ds(0, L)]
        for k in range(HID):
            w = wdec[pl.ds(k * OUT_P, L)]
            lg = lg + w * jnp.broadcast_to(fv[k // L][k % L], (L,))
        ls = [lg[i] for i in range(OUT_F)]
        mx = ls[0]
        for i in range(1, OUT_F):
            mx = jnp.maximum(mx, ls[i])
        sh = lg - jnp.broadcast_to(mx, (L,))
        ex = jnp.exp(sh)
        ssum = ex[0]
        for i in range(1, OUT_F):
            ssum = ssum + ex[i]
        ssv = jnp.broadcast_to(ssum, (L,))
        yv = jnp.full((L,), 1.2, jnp.float32)
        for _ in range(8):
            yv = yv + ssv * jnp.exp(-yv) - 1.0
        fin[pl.ds(j * OUT_P, L)] = sh - yv

    pltpu.sync_copy(fin, out_hbm.at[pl.ds(s0 * OUT_P, SIMS * OUT_P)])


def kernel(x, edge_index, nodes, parents, first_message,
           W_enc, b_enc, W_ns, b_ns, W_nm, b_nm, W_dec, b_dec):
    del edge_index
    f32 = jnp.float32

    sim = pl.kernel(
        _sim_body,
        out_type=jax.ShapeDtypeStruct((S * OUT_P,), f32),
        mesh=plsc.VectorSubcoreMesh(core_axis_name="c", subcore_axis_name="s"),
        scratch_types=[
            pltpu.VMEM((N * HID,), f32),
            pltpu.VMEM((N * HID,), f32),
            pltpu.VMEM((T * MSG,), f32),
            pltpu.VMEM((T * MSG,), f32),
            pltpu.VMEM((4 * T,), jnp.int32),
            pltpu.SMEM((4 * T,), jnp.int32),
            pltpu.VMEM(((HID + MSG) * HID,), f32),
            pltpu.VMEM(((HID + MSG) * MSG,), f32),
            pltpu.VMEM((HID,), f32),
            pltpu.VMEM((MSG,), f32),
            pltpu.VMEM((MSG,), f32),
            pltpu.VMEM((ROWS * IN_F,), f32),
            pltpu.VMEM((IN_F * HID,), f32),
            pltpu.VMEM((HID,), f32),
            pltpu.VMEM((HID * OUT_P,), f32),
            pltpu.VMEM((OUT_P,), f32),
            pltpu.VMEM((ROWS * HID,), f32),
            pltpu.VMEM((SIMS * OUT_P,), f32),
            pltpu.VMEM_SHARED((N * HID,), f32),
            pltpu.SemaphoreType.DMA,
        ],
    )

    wdec_p = jnp.pad(W_dec.T, ((0, 0), (0, OUT_P - OUT_F)))
    bdec_p = jnp.pad(b_dec, (0, OUT_P - OUT_F))
    out = sim(x.reshape(N * IN_F),
              nodes.astype(jnp.int32).reshape(S * T),
              parents.astype(jnp.int32).reshape(S * T),
              first_message.reshape(MSG),
              W_enc.T.reshape(IN_F * HID), b_enc,
              W_ns.T.reshape((HID + MSG) * HID), b_ns,
              W_nm.T.reshape((HID + MSG) * MSG), b_nm,
              wdec_p.reshape(HID * OUT_P), bdec_p)
    return out.reshape(S, OUT_P)[:, :OUT_F]


# speedup vs baseline: 2.0922x; 1.0002x over previous
import jax
import jax.numpy as jnp
from jax import lax
from jax.experimental import pallas as pl
from jax.experimental.pallas import tpu as pltpu
from jax.experimental.pallas import tpu_sc as plsc

N = 64
S = 64
T = 320
IN_F = 128
HID = 64
MSG = 32
OUT_F = 10
OUT_P = 16
NC = 2
NS = 16
NW = NC * NS
SIMS = S // NW
ROWS = N // NS
L = 16


def _sim_body(x_hbm, nodes_hbm, parents_hbm, first_hbm,
              wenc_hbm, benc_hbm, wns_hbm, bns_hbm, wnm_hbm, bnm_hbm,
              wdec_hbm, bdec_hbm, out_hbm,
              pred0, pred1, msgs0, msgs1, npbuf, nps,
              wns, wnm, bns, bnm, fst,
              xbuf, wenc, benc, wdec, bdec, encstage, fin, enc_sh, dma_sem):
    cid = lax.axis_index("c")
    tid = lax.axis_index("s")
    wid = tid * NC + cid
    s0 = wid * SIMS

    handles = [
        pltpu.async_copy(x_hbm.at[pl.ds(tid * ROWS * IN_F, ROWS * IN_F)],
                         xbuf, dma_sem),
        pltpu.async_copy(wenc_hbm, wenc, dma_sem),
        pltpu.async_copy(benc_hbm, benc, dma_sem),
        pltpu.async_copy(nodes_hbm.at[pl.ds(s0 * T, T)],
                         npbuf.at[pl.ds(0, T)], dma_sem),
        pltpu.async_copy(nodes_hbm.at[pl.ds((s0 + 1) * T, T)],
                         npbuf.at[pl.ds(T, T)], dma_sem),
        pltpu.async_copy(parents_hbm.at[pl.ds(s0 * T, T)],
                         npbuf.at[pl.ds(2 * T, T)], dma_sem),
        pltpu.async_copy(parents_hbm.at[pl.ds((s0 + 1) * T, T)],
                         npbuf.at[pl.ds(3 * T, T)], dma_sem),
        pltpu.async_copy(wns_hbm, wns, dma_sem),
        pltpu.async_copy(wnm_hbm, wnm, dma_sem),
        pltpu.async_copy(bns_hbm, bns, dma_sem),
        pltpu.async_copy(bnm_hbm, bnm, dma_sem),
        pltpu.async_copy(first_hbm, fst, dma_sem),
        pltpu.async_copy(wdec_hbm, wdec, dma_sem),
        pltpu.async_copy(bdec_hbm, bdec, dma_sem),
    ]
    for h in handles:
        h.wait()

    for b in range(4 * T // L):
        v = npbuf[pl.ds(b * L, L)]
        for i in range(L):
            nps[b * L + i] = v[i]

    xv = [[xbuf[pl.ds(r * IN_F + kb * L, L)] for kb in range(IN_F // L)]
          for r in range(ROWS)]
    acc = [[benc[pl.ds(hb * L, L)] for hb in range(4)] for _ in range(ROWS)]
    for k in range(IN_F):
        wcol = [wenc[pl.ds(k * HID + hb * L, L)] for hb in range(4)]
        for r in range(ROWS):
            xk = jnp.broadcast_to(xv[r][k // L][k % L], (L,))
            for hb in range(4):
                acc[r][hb] = acc[r][hb] + wcol[hb] * xk
    for r in range(ROWS):
        for hb in range(4):
            encstage[pl.ds(r * HID + hb * L, L)] = acc[r][hb]

    pltpu.sync_copy(encstage, enc_sh.at[pl.ds(tid * ROWS * HID, ROWS * HID)])
    plsc.subcore_barrier()
    pltpu.sync_copy(enc_sh, pred0)
    pltpu.sync_copy(enc_sh, pred1)

    fstv = [fst[pl.ds(0, L)], fst[pl.ds(L, L)]]

    def step(t, carry):
        nd0 = nps[t]
        nd1 = nps[T + t]
        pa0 = nps[2 * T + t]
        pa1 = nps[3 * T + t]
        uf0 = pa0 < 0
        uf1 = pa1 < 0
        p0 = jnp.maximum(pa0, 0)
        p1 = jnp.maximum(pa1, 0)

        zero = jnp.zeros((L,), jnp.float32)
        a0 = [[bns[pl.ds(hb * L, L)] for hb in range(4)], [zero] * 4]
        a1 = [[bns[pl.ds(hb * L, L)] for hb in range(4)], [zero] * 4]
        m0 = [[bnm[pl.ds(mb * L, L)] for mb in range(2)], [zero] * 2]
        m1 = [[bnm[pl.ds(mb * L, L)] for mb in range(2)], [zero] * 2]

        iv0 = [pred0[pl.ds(nd0 * HID + hb * L, L)] for hb in range(4)]
        iv1 = [pred1[pl.ds(nd1 * HID + hb * L, L)] for hb in range(4)]
        iv0 += [jnp.where(uf0, fstv[mb], msgs0[pl.ds(p0 * MSG + mb * L, L)])
                for mb in range(2)]
        iv1 += [jnp.where(uf1, fstv[mb], msgs1[pl.ds(p1 * MSG + mb * L, L)])
                for mb in range(2)]

        for k in range(HID + MSG):
            v0 = jnp.broadcast_to(iv0[k // L][k % L], (L,))
            v1 = jnp.broadcast_to(iv1[k // L][k % L], (L,))
            par = k % 2
            for hb in range(4):
                w = wns[pl.ds(k * HID + hb * L, L)]
                a0[par][hb] = a0[par][hb] + w * v0
                a1[par][hb] = a1[par][hb] + w * v1
            if k >= HID:
                for mb in range(2):
                    w = wnm[pl.ds(k * MSG + mb * L, L)]
                    m0[par][mb] = m0[par][mb] + w * v0
                    m1[par][mb] = m1[par][mb] + w * v1

        ns0 = [jnp.maximum(a0[0][hb] + a0[1][hb], 0.0) for hb in range(4)]
        ns1 = [jnp.maximum(a1[0][hb] + a1[1][hb], 0.0) for hb in range(4)]
        for hb in range(4):
            pred0[pl.ds(nd0 * HID + hb * L, L)] = ns0[hb]
            pred1[pl.ds(nd1 * HID + hb * L, L)] = ns1[hb]

        for k in range(HID):
            n0 = jnp.broadcast_to(ns0[k // L][k % L], (L,))
            n1 = jnp.broadcast_to(ns1[k // L][k % L], (L,))
            par = k % 2
            for mb in range(2):
                w = wnm[pl.ds(k * MSG + mb * L, L)]
                m0[par][mb] = m0[par][mb] + w * n0
                m1[par][mb] = m1[par][mb] + w * n1
        m0 = [m0[0][mb] + m0[1][mb] for mb in range(2)]
        m1 = [m1[0][mb] + m1[1][mb] for mb in range(2)]

        for mb in range(2):
            msgs0[pl.ds(t * MSG + mb * L, L)] = m0[mb]
            msgs1[pl.ds(t * MSG + mb * L, L)] = m1[mb]
        return carry

    lax.fori_loop(0, T, step, 0)

    for j, (predj, sg) in enumerate(((pred0, s0), (pred1, s0 + 1))):
        fv = [predj[pl.ds(sg * HID + hb * L, L)] for hb in range(4)]
        lg = bdec[pl.ds(0, L)]
        for k in range(HID):
            w = wdec[pl.ds(k * OUT_P, L)]
            lg = lg + w * jnp.broadcast_to(fv[k // L][k % L], (L,))
        ls = [lg[i] for i in range(OUT_F)]
        mx = ls[0]
        for i in range(1, OUT_F):
            mx = jnp.maximum(mx, ls[i])
        sh = lg - jnp.broadcast_to(mx, (L,))
        ex = jnp.exp(sh)
        ssum = ex[0]
        for i in range(1, OUT_F):
            ssum = ssum + ex[i]
        ssv = jnp.broadcast_to(ssum, (L,))
        yv = jnp.full((L,), 1.2, jnp.float32)
        for _ in range(8):
            yv = yv + ssv * jnp.exp(-yv) - 1.0
        fin[pl.ds(j * OUT_P, L)] = sh - yv

    pltpu.sync_copy(fin, out_hbm.at[pl.ds(s0 * OUT_P, SIMS * OUT_P)])


def kernel(x, edge_index, nodes, parents, first_message,
           W_enc, b_enc, W_ns, b_ns, W_nm, b_nm, W_dec, b_dec):
    del edge_index
    f32 = jnp.float32

    sim = pl.kernel(
        _sim_body,
        out_type=jax.ShapeDtypeStruct((S * OUT_P,), f32),
        mesh=plsc.VectorSubcoreMesh(core_axis_name="c", subcore_axis_name="s"),
        scratch_types=[
            pltpu.VMEM((N * HID,), f32),
            pltpu.VMEM((N * HID,), f32),
            pltpu.VMEM((T * MSG,), f32),
            pltpu.VMEM((T * MSG,), f32),
            pltpu.VMEM((4 * T,), jnp.int32),
            pltpu.SMEM((4 * T,), jnp.int32),
            pltpu.VMEM(((HID + MSG) * HID,), f32),
            pltpu.VMEM(((HID + MSG) * MSG,), f32),
            pltpu.VMEM((HID,), f32),
            pltpu.VMEM((MSG,), f32),
            pltpu.VMEM((MSG,), f32),
            pltpu.VMEM((ROWS * IN_F,), f32),
            pltpu.VMEM((IN_F * HID,), f32),
            pltpu.VMEM((HID,), f32),
            pltpu.VMEM((HID * OUT_P,), f32),
            pltpu.VMEM((OUT_P,), f32),
            pltpu.VMEM((ROWS * HID,), f32),
            pltpu.VMEM((SIMS * OUT_P,), f32),
            pltpu.VMEM_SHARED((N * HID,), f32),
            pltpu.SemaphoreType.DMA,
        ],
    )

    wdec_p = jnp.pad(W_dec.T, ((0, 0), (0, OUT_P - OUT_F)))
    bdec_p = jnp.pad(b_dec, (0, OUT_P - OUT_F))
    out = sim(x.reshape(N * IN_F),
              nodes.astype(jnp.int32).reshape(S * T),
              parents.astype(jnp.int32).reshape(S * T),
              first_message.reshape(MSG),
              W_enc.T.reshape(IN_F * HID), b_enc,
              W_ns.T.reshape((HID + MSG) * HID), b_ns,
              W_nm.T.reshape((HID + MSG) * MSG), b_nm,
              wdec_p.reshape(HID * OUT_P), bdec_p)
    return out.reshape(S, OUT_P)[:, :OUT_F]
